# baseline (speedup 1.0000x reference)
import functools

import jax
import jax.numpy as jnp
from jax import lax
from jax.experimental import pallas as pl
from jax.experimental.pallas import tpu as pltpu
from jax.experimental.pallas import tpu_sc as plsc

EMBED_DIM = 128
NUM_CORES = 2
NUM_SUBCORES = 16
NUM_WORKERS = NUM_CORES * NUM_SUBCORES
CHUNK = 400
NBUF = 2


def _gather_sc(table, flat_ids):
    n = flat_ids.shape[0]
    per_worker = n // NUM_WORKERS
    nchunks = per_worker // CHUNK
    assert per_worker % CHUNK == 0 and nchunks % NBUF == 0
    mesh = plsc.VectorSubcoreMesh(core_axis_name="c", subcore_axis_name="s")

    @functools.partial(
        pl.kernel,
        mesh=mesh,
        out_type=jax.ShapeDtypeStruct((n, EMBED_DIM), table.dtype),
        scratch_types=(
            [pltpu.VMEM((CHUNK,), jnp.int32) for _ in range(NBUF)]
            + [
                pltpu.VMEM((NBUF, CHUNK, EMBED_DIM), jnp.float32),
                pltpu.SemaphoreType.DMA((NBUF,)),
                pltpu.SemaphoreType.DMA((NBUF,)),
                pltpu.SemaphoreType.DMA((NBUF,)),
            ]
        ),
    )
    def gather_kernel(table_hbm, ids_hbm, out_hbm, ib0, ib1, bufs,
                      isems, gsems, wsems):
        ibufs = [ib0, ib1]
        wid = lax.axis_index("s") * NUM_CORES + lax.axis_index("c")

        def row0(c):
            return (c * NUM_WORKERS + wid) * CHUNK

        def start_idx(c, b):
            pltpu.async_copy(ids_hbm.at[pl.ds(row0(c), CHUNK)],
                             ibufs[b], isems.at[b])

        def wait_idx(c, b):
            pltpu.make_async_copy(ids_hbm.at[pl.ds(row0(c), CHUNK)],
                                  ibufs[b], isems.at[b]).wait()

        def start_gather(b):
            pltpu.async_copy(table_hbm.at[ibufs[b]], bufs.at[b],
                             gsems.at[b])

        def wait_gather(b):
            pltpu.make_async_copy(table_hbm.at[ibufs[b]], bufs.at[b],
                                  gsems.at[b]).wait()

        def start_write(c, b):
            pltpu.async_copy(bufs.at[b], out_hbm.at[pl.ds(row0(c), CHUNK)],
                             wsems.at[b])

        def wait_write(c, b):
            pltpu.make_async_copy(bufs.at[b],
                                  out_hbm.at[pl.ds(row0(c), CHUNK)],
                                  wsems.at[b]).wait()

        for b in range(NBUF):
            start_idx(b, b)
        for b in range(NBUF):
            wait_idx(b, b)
            start_gather(b)

        @pl.loop(0, nchunks, step=NBUF)
        def _(g):
            for b in range(NBUF):
                wait_gather(b)

                @pl.when(g + b + NBUF < nchunks)
                def _():
                    start_idx(g + b + NBUF, b)

                start_write(g + b, b)
            for b in range(NBUF):
                @pl.when(g + b + NBUF < nchunks)
                def _():
                    wait_write(g + b, b)
                    wait_idx(g + b + NBUF, b)
                    start_gather(b)

        for b in range(NBUF):
            wait_write(nchunks - NBUF + b, b)

    return gather_kernel(table, flat_ids)


def kernel(token_ids, table):
    batch, seq = token_ids.shape
    flat = token_ids.reshape(batch * seq).astype(jnp.int32)
    out = _gather_sc(table, flat)
    return out.reshape(batch, seq, EMBED_DIM)


# speedup vs baseline: 1.0351x; 1.0351x over previous
import functools

import jax
import jax.numpy as jnp
from jax import lax
from jax.experimental import pallas as pl
from jax.experimental.pallas import tpu as pltpu
from jax.experimental.pallas import tpu_sc as plsc

EMBED_DIM = 128
NUM_CORES = 2
NUM_SUBCORES = 16
NUM_WORKERS = NUM_CORES * NUM_SUBCORES
CHUNK = 160
NBUF = 5


def _gather_sc(table, flat_ids):
    n = flat_ids.shape[0]
    per_worker = n // NUM_WORKERS
    nchunks = per_worker // CHUNK
    assert per_worker % CHUNK == 0 and nchunks % NBUF == 0
    mesh = plsc.VectorSubcoreMesh(core_axis_name="c", subcore_axis_name="s")

    @functools.partial(
        pl.kernel,
        mesh=mesh,
        out_type=jax.ShapeDtypeStruct((n, EMBED_DIM), table.dtype),
        scratch_types=(
            [pltpu.VMEM((CHUNK,), jnp.int32) for _ in range(NBUF)]
            + [
                pltpu.VMEM((NBUF, CHUNK, EMBED_DIM), jnp.float32),
                pltpu.SemaphoreType.DMA((NBUF,)),
                pltpu.SemaphoreType.DMA((NBUF,)),
                pltpu.SemaphoreType.DMA((NBUF,)),
            ]
        ),
    )
    def gather_kernel(table_hbm, ids_hbm, out_hbm, ib0, ib1, ib2, ib3,
                      ib4, bufs, isems, gsems, wsems):
        ibufs = [ib0, ib1, ib2, ib3, ib4]
        wid = lax.axis_index("s") * NUM_CORES + lax.axis_index("c")

        def row0(c):
            return (c * NUM_WORKERS + wid) * CHUNK

        def start_idx(c, b):
            pltpu.async_copy(ids_hbm.at[pl.ds(row0(c), CHUNK)],
                             ibufs[b], isems.at[b])

        def wait_idx(c, b):
            pltpu.make_async_copy(ids_hbm.at[pl.ds(row0(c), CHUNK)],
                                  ibufs[b], isems.at[b]).wait()

        def start_gather(b):
            pltpu.async_copy(table_hbm.at[ibufs[b]], bufs.at[b],
                             gsems.at[b])

        def wait_gather(b):
            pltpu.make_async_copy(table_hbm.at[ibufs[b]], bufs.at[b],
                                  gsems.at[b]).wait()

        def start_write(c, b):
            pltpu.async_copy(bufs.at[b], out_hbm.at[pl.ds(row0(c), CHUNK)],
                             wsems.at[b])

        def wait_write(c, b):
            pltpu.make_async_copy(bufs.at[b],
                                  out_hbm.at[pl.ds(row0(c), CHUNK)],
                                  wsems.at[b]).wait()

        for b in range(NBUF):
            start_idx(b, b)
        for b in range(NBUF):
            wait_idx(b, b)
            start_gather(b)

        @pl.loop(0, nchunks, step=NBUF)
        def _(g):
            for b in range(NBUF):
                wait_gather(b)

                @pl.when(g + b + NBUF < nchunks)
                def _():
                    start_idx(g + b + NBUF, b)

                start_write(g + b, b)
            for b in range(NBUF):
                @pl.when(g + b + NBUF < nchunks)
                def _():
                    wait_write(g + b, b)
                    wait_idx(g + b + NBUF, b)
                    start_gather(b)

        for b in range(NBUF):
            wait_write(nchunks - NBUF + b, b)

    return gather_kernel(table, flat_ids)


def kernel(token_ids, table):
    batch, seq = token_ids.shape
    flat = token_ids.reshape(batch * seq).astype(jnp.int32)
    out = _gather_sc(table, flat)
    return out.reshape(batch, seq, EMBED_DIM)
